# baseline (speedup 1.0000x reference)
import jax
import jax.numpy as jnp
from jax.experimental import pallas as pl
from jax.experimental.pallas import tpu as pltpu

_B, _H, _W, _D = 8, 128, 128, 64
_HC = 16
_NH = _H // _HC


def _sums_body(yt0_ref, yp_ref, acc1_ref, acc2_ref, cnt_ref, m1_ref, m2_ref):
    b = pl.program_id(0)
    hc = pl.program_id(1)

    @pl.when(jnp.logical_and(b == 0, hc == 0))
    def _zero():
        acc1_ref[...] = jnp.zeros((_H, _W), jnp.float32)
        acc2_ref[...] = jnp.zeros((_H, _W), jnp.float32)
        cnt_ref[...] = jnp.zeros((1, 128), jnp.float32)

    @pl.when(hc == 0)
    def _masks():
        s0 = yt0_ref[0, 0]
        m1 = jnp.logical_and(s0 > 1000.0, s0 < 3000.0).astype(jnp.float32)
        m2 = jnp.logical_or(
            jnp.logical_and(s0 > 0.0, s0 < 1000.0), s0 > 3000.0
        ).astype(jnp.float32)
        m1_ref[...] = m1
        m2_ref[...] = m2
        lane = jax.lax.broadcasted_iota(jnp.int32, (1, 128), 1)
        upd = jnp.where(lane == 0, jnp.sum(m1), 0.0) + jnp.where(
            lane == 1, jnp.sum(m2), 0.0
        )
        cnt_ref[...] += upd

    yp = yp_ref[0]
    m1 = m1_ref[...]
    m2 = m2_ref[...]
    p1 = jnp.sum(yp * m1[None], axis=1)
    p2 = jnp.sum(yp * m2[None], axis=1)

    sl = pl.ds(hc * _HC, _HC)
    acc1_ref[sl, :] += p1
    acc2_ref[sl, :] += p2


def _corr_body(acc1_ref, acc2_ref, cnt_ref, out_ref):
    a = jnp.sum(acc1_ref[...], axis=1, keepdims=True) / cnt_ref[0, 0]
    bb = jnp.sum(acc2_ref[...], axis=1, keepdims=True) / cnt_ref[0, 1]
    am = a - jnp.mean(a)
    bm = bb - jnp.mean(bb)
    cov = jnp.mean(am * bm)
    sx = jnp.sqrt(jnp.mean(am * am))
    sy = jnp.sqrt(jnp.mean(bm * bm))
    corr = cov / (sx * sy)
    out_ref[...] = jnp.abs(jnp.clip(corr, -1.0, 1.0)).reshape(1, 1)


def kernel(y_true, y_pred):
    yt = jnp.transpose(y_true[..., 0], (0, 1, 3, 2))
    yp = jnp.transpose(y_pred[..., 0], (0, 1, 3, 2))
    acc1, acc2, cnt = pl.pallas_call(
        _sums_body,
        grid=(_B, _NH),
        in_specs=[
            pl.BlockSpec((1, 1, _D, _W), lambda b, hc: (b, 0, 0, 0)),
            pl.BlockSpec((1, _HC, _D, _W), lambda b, hc: (b, hc, 0, 0)),
        ],
        out_specs=[
            pl.BlockSpec((_H, _W), lambda b, hc: (0, 0)),
            pl.BlockSpec((_H, _W), lambda b, hc: (0, 0)),
            pl.BlockSpec((1, 128), lambda b, hc: (0, 0)),
        ],
        out_shape=[
            jax.ShapeDtypeStruct((_H, _W), jnp.float32),
            jax.ShapeDtypeStruct((_H, _W), jnp.float32),
            jax.ShapeDtypeStruct((1, 128), jnp.float32),
        ],
        scratch_shapes=[
            pltpu.VMEM((_D, _W), jnp.float32),
            pltpu.VMEM((_D, _W), jnp.float32),
        ],
    )(yt, yp)
    out = pl.pallas_call(
        _corr_body,
        out_shape=jax.ShapeDtypeStruct((1, 1), jnp.float32),
    )(acc1, acc2, cnt)
    return out


# speedup vs baseline: 1.5271x; 1.5271x over previous
import jax
import jax.numpy as jnp
from jax.experimental import pallas as pl
from jax.experimental.pallas import tpu as pltpu

_B, _H, _W, _D = 8, 128, 128, 64
_HC = 32
_NH = _H // _HC


def _sums_body(yt0_ref, yp_ref, acc1_ref, acc2_ref, cnt_ref, m1_ref, m2_ref):
    b = pl.program_id(0)
    hc = pl.program_id(1)

    @pl.when(jnp.logical_and(b == 0, hc == 0))
    def _zero():
        acc1_ref[...] = jnp.zeros((_H, _W), jnp.float32)
        acc2_ref[...] = jnp.zeros((_H, _W), jnp.float32)
        cnt_ref[...] = jnp.zeros((1, 128), jnp.float32)

    @pl.when(hc == 0)
    def _masks():
        s0 = yt0_ref[0, 0]
        m1 = jnp.logical_and(s0 > 1000.0, s0 < 3000.0).astype(jnp.float32)
        m2 = jnp.logical_or(
            jnp.logical_and(s0 > 0.0, s0 < 1000.0), s0 > 3000.0
        ).astype(jnp.float32)
        m1_ref[...] = m1
        m2_ref[...] = m2
        lane = jax.lax.broadcasted_iota(jnp.int32, (1, 128), 1)
        upd = jnp.where(lane == 0, jnp.sum(m1), 0.0) + jnp.where(
            lane == 1, jnp.sum(m2), 0.0
        )
        cnt_ref[...] += upd

    yp = yp_ref[0]
    m1 = m1_ref[...]
    m2 = m2_ref[...]
    p1 = jnp.sum(yp * m1[None], axis=1)
    p2 = jnp.sum(yp * m2[None], axis=1)

    sl = pl.ds(hc * _HC, _HC)
    acc1_ref[sl, :] += p1
    acc2_ref[sl, :] += p2


def _corr_body(acc1_ref, acc2_ref, cnt_ref, out_ref):
    a = jnp.sum(acc1_ref[...], axis=1, keepdims=True) / cnt_ref[0, 0]
    bb = jnp.sum(acc2_ref[...], axis=1, keepdims=True) / cnt_ref[0, 1]
    am = a - jnp.mean(a)
    bm = bb - jnp.mean(bb)
    cov = jnp.mean(am * bm)
    sx = jnp.sqrt(jnp.mean(am * am))
    sy = jnp.sqrt(jnp.mean(bm * bm))
    corr = cov / (sx * sy)
    out_ref[...] = jnp.abs(jnp.clip(corr, -1.0, 1.0)).reshape(1, 1)


def kernel(y_true, y_pred):
    yt = jnp.transpose(y_true[..., 0], (0, 1, 3, 2))
    yp = jnp.transpose(y_pred[..., 0], (0, 1, 3, 2))
    acc1, acc2, cnt = pl.pallas_call(
        _sums_body,
        grid=(_B, _NH),
        in_specs=[
            pl.BlockSpec((1, 1, _D, _W), lambda b, hc: (b, 0, 0, 0)),
            pl.BlockSpec((1, _HC, _D, _W), lambda b, hc: (b, hc, 0, 0)),
        ],
        out_specs=[
            pl.BlockSpec((_H, _W), lambda b, hc: (0, 0)),
            pl.BlockSpec((_H, _W), lambda b, hc: (0, 0)),
            pl.BlockSpec((1, 128), lambda b, hc: (0, 0)),
        ],
        out_shape=[
            jax.ShapeDtypeStruct((_H, _W), jnp.float32),
            jax.ShapeDtypeStruct((_H, _W), jnp.float32),
            jax.ShapeDtypeStruct((1, 128), jnp.float32),
        ],
        scratch_shapes=[
            pltpu.VMEM((_D, _W), jnp.float32),
            pltpu.VMEM((_D, _W), jnp.float32),
        ],
    )(yt, yp)
    out = pl.pallas_call(
        _corr_body,
        out_shape=jax.ShapeDtypeStruct((1, 1), jnp.float32),
    )(acc1, acc2, cnt)
    return out


# speedup vs baseline: 2.0960x; 1.3726x over previous
import jax
import jax.numpy as jnp
from jax.experimental import pallas as pl
from jax.experimental.pallas import tpu as pltpu

_B, _H, _W, _D = 8, 128, 128, 64
_HC = 64
_NH = _H // _HC


def _sums_body(yt0_ref, yp_ref, acc1_ref, acc2_ref, cnt_ref, m1_ref, m2_ref):
    b = pl.program_id(0)
    hc = pl.program_id(1)

    @pl.when(jnp.logical_and(b == 0, hc == 0))
    def _zero():
        acc1_ref[...] = jnp.zeros((_H, _W), jnp.float32)
        acc2_ref[...] = jnp.zeros((_H, _W), jnp.float32)
        cnt_ref[...] = jnp.zeros((1, 128), jnp.float32)

    @pl.when(hc == 0)
    def _masks():
        s0 = yt0_ref[0, 0]
        m1 = jnp.logical_and(s0 > 1000.0, s0 < 3000.0).astype(jnp.float32)
        m2 = jnp.logical_or(
            jnp.logical_and(s0 > 0.0, s0 < 1000.0), s0 > 3000.0
        ).astype(jnp.float32)
        m1_ref[...] = m1
        m2_ref[...] = m2
        lane = jax.lax.broadcasted_iota(jnp.int32, (1, 128), 1)
        upd = jnp.where(lane == 0, jnp.sum(m1), 0.0) + jnp.where(
            lane == 1, jnp.sum(m2), 0.0
        )
        cnt_ref[...] += upd

    yp = yp_ref[0]
    m1 = m1_ref[...]
    m2 = m2_ref[...]
    p1 = jnp.sum(yp * m1[None], axis=1)
    p2 = jnp.sum(yp * m2[None], axis=1)

    sl = pl.ds(hc * _HC, _HC)
    acc1_ref[sl, :] += p1
    acc2_ref[sl, :] += p2


def _corr_body(acc1_ref, acc2_ref, cnt_ref, out_ref):
    a = jnp.sum(acc1_ref[...], axis=1, keepdims=True) / cnt_ref[0, 0]
    bb = jnp.sum(acc2_ref[...], axis=1, keepdims=True) / cnt_ref[0, 1]
    am = a - jnp.mean(a)
    bm = bb - jnp.mean(bb)
    cov = jnp.mean(am * bm)
    sx = jnp.sqrt(jnp.mean(am * am))
    sy = jnp.sqrt(jnp.mean(bm * bm))
    corr = cov / (sx * sy)
    out_ref[...] = jnp.abs(jnp.clip(corr, -1.0, 1.0)).reshape(1, 1)


def kernel(y_true, y_pred):
    yt = jnp.transpose(y_true[..., 0], (0, 1, 3, 2))
    yp = jnp.transpose(y_pred[..., 0], (0, 1, 3, 2))
    acc1, acc2, cnt = pl.pallas_call(
        _sums_body,
        grid=(_B, _NH),
        in_specs=[
            pl.BlockSpec((1, 1, _D, _W), lambda b, hc: (b, 0, 0, 0)),
            pl.BlockSpec((1, _HC, _D, _W), lambda b, hc: (b, hc, 0, 0)),
        ],
        out_specs=[
            pl.BlockSpec((_H, _W), lambda b, hc: (0, 0)),
            pl.BlockSpec((_H, _W), lambda b, hc: (0, 0)),
            pl.BlockSpec((1, 128), lambda b, hc: (0, 0)),
        ],
        out_shape=[
            jax.ShapeDtypeStruct((_H, _W), jnp.float32),
            jax.ShapeDtypeStruct((_H, _W), jnp.float32),
            jax.ShapeDtypeStruct((1, 128), jnp.float32),
        ],
        scratch_shapes=[
            pltpu.VMEM((_D, _W), jnp.float32),
            pltpu.VMEM((_D, _W), jnp.float32),
        ],
    )(yt, yp)
    out = pl.pallas_call(
        _corr_body,
        out_shape=jax.ShapeDtypeStruct((1, 1), jnp.float32),
    )(acc1, acc2, cnt)
    return out


# speedup vs baseline: 2.4352x; 1.1618x over previous
import jax
import jax.numpy as jnp
from jax.experimental import pallas as pl
from jax.experimental.pallas import tpu as pltpu

_B, _H, _W, _D = 8, 128, 128, 64
_HC = 128
_NH = _H // _HC


def _sums_body(yt0_ref, yp_ref, acc1_ref, acc2_ref, cnt_ref, m1_ref, m2_ref):
    b = pl.program_id(0)
    hc = pl.program_id(1)

    @pl.when(jnp.logical_and(b == 0, hc == 0))
    def _zero():
        acc1_ref[...] = jnp.zeros((_H, _W), jnp.float32)
        acc2_ref[...] = jnp.zeros((_H, _W), jnp.float32)
        cnt_ref[...] = jnp.zeros((1, 128), jnp.float32)

    @pl.when(hc == 0)
    def _masks():
        s0 = yt0_ref[0, 0]
        m1 = jnp.logical_and(s0 > 1000.0, s0 < 3000.0).astype(jnp.float32)
        m2 = jnp.logical_or(
            jnp.logical_and(s0 > 0.0, s0 < 1000.0), s0 > 3000.0
        ).astype(jnp.float32)
        m1_ref[...] = m1
        m2_ref[...] = m2
        lane = jax.lax.broadcasted_iota(jnp.int32, (1, 128), 1)
        upd = jnp.where(lane == 0, jnp.sum(m1), 0.0) + jnp.where(
            lane == 1, jnp.sum(m2), 0.0
        )
        cnt_ref[...] += upd

    yp = yp_ref[0]
    m1 = m1_ref[...]
    m2 = m2_ref[...]
    p1 = jnp.sum(yp * m1[None], axis=1)
    p2 = jnp.sum(yp * m2[None], axis=1)

    sl = pl.ds(hc * _HC, _HC)
    acc1_ref[sl, :] += p1
    acc2_ref[sl, :] += p2


def _corr_body(acc1_ref, acc2_ref, cnt_ref, out_ref):
    a = jnp.sum(acc1_ref[...], axis=1, keepdims=True) / cnt_ref[0, 0]
    bb = jnp.sum(acc2_ref[...], axis=1, keepdims=True) / cnt_ref[0, 1]
    am = a - jnp.mean(a)
    bm = bb - jnp.mean(bb)
    cov = jnp.mean(am * bm)
    sx = jnp.sqrt(jnp.mean(am * am))
    sy = jnp.sqrt(jnp.mean(bm * bm))
    corr = cov / (sx * sy)
    out_ref[...] = jnp.abs(jnp.clip(corr, -1.0, 1.0)).reshape(1, 1)


def kernel(y_true, y_pred):
    yt = jnp.transpose(y_true[..., 0], (0, 1, 3, 2))
    yp = jnp.transpose(y_pred[..., 0], (0, 1, 3, 2))
    acc1, acc2, cnt = pl.pallas_call(
        _sums_body,
        grid=(_B, _NH),
        in_specs=[
            pl.BlockSpec((1, 1, _D, _W), lambda b, hc: (b, 0, 0, 0)),
            pl.BlockSpec((1, _HC, _D, _W), lambda b, hc: (b, hc, 0, 0)),
        ],
        out_specs=[
            pl.BlockSpec((_H, _W), lambda b, hc: (0, 0)),
            pl.BlockSpec((_H, _W), lambda b, hc: (0, 0)),
            pl.BlockSpec((1, 128), lambda b, hc: (0, 0)),
        ],
        out_shape=[
            jax.ShapeDtypeStruct((_H, _W), jnp.float32),
            jax.ShapeDtypeStruct((_H, _W), jnp.float32),
            jax.ShapeDtypeStruct((1, 128), jnp.float32),
        ],
        scratch_shapes=[
            pltpu.VMEM((_D, _W), jnp.float32),
            pltpu.VMEM((_D, _W), jnp.float32),
        ],
    )(yt, yp)
    out = pl.pallas_call(
        _corr_body,
        out_shape=jax.ShapeDtypeStruct((1, 1), jnp.float32),
    )(acc1, acc2, cnt)
    return out


# speedup vs baseline: 2.5506x; 1.0474x over previous
import jax
import jax.numpy as jnp
from jax.experimental import pallas as pl
from jax.experimental.pallas import tpu as pltpu

_B, _H, _W, _D = 8, 128, 128, 64
_NS = 2
_NB = _B // _NS


def _masks_of(s0):
    m1 = jnp.logical_and(s0 > 1000.0, s0 < 3000.0).astype(jnp.float32)
    m2 = jnp.logical_or(
        jnp.logical_and(s0 > 0.0, s0 < 1000.0), s0 > 3000.0
    ).astype(jnp.float32)
    return m1, m2


def _sums_body(yt_a_ref, yt_b_ref, yp_a_ref, yp_b_ref,
               acc1_ref, acc2_ref, cnt_ref):
    b = pl.program_id(0)

    @pl.when(b == 0)
    def _zero():
        acc1_ref[...] = jnp.zeros((_H, _W), jnp.float32)
        acc2_ref[...] = jnp.zeros((_H, _W), jnp.float32)
        cnt_ref[...] = jnp.zeros((1, 128), jnp.float32)

    m1a, m2a = _masks_of(yt_a_ref[0, 0])
    m1b, m2b = _masks_of(yt_b_ref[0, 0])

    ypa = yp_a_ref[0]
    ypb = yp_b_ref[0]
    p1 = jnp.sum(ypa * m1a[None], axis=1) + jnp.sum(ypb * m1b[None], axis=1)
    p2 = jnp.sum(ypa * m2a[None], axis=1) + jnp.sum(ypb * m2b[None], axis=1)

    acc1_ref[...] += p1
    acc2_ref[...] += p2

    lane = jax.lax.broadcasted_iota(jnp.int32, (1, 128), 1)
    c1 = jnp.sum(m1a) + jnp.sum(m1b)
    c2 = jnp.sum(m2a) + jnp.sum(m2b)
    cnt_ref[...] += jnp.where(lane == 0, c1, 0.0) + jnp.where(lane == 1, c2, 0.0)


def _corr_body(acc1_ref, acc2_ref, cnt_ref, out_ref):
    a = jnp.sum(acc1_ref[...], axis=1, keepdims=True) / cnt_ref[0, 0]
    bb = jnp.sum(acc2_ref[...], axis=1, keepdims=True) / cnt_ref[0, 1]
    am = a - jnp.mean(a)
    bm = bb - jnp.mean(bb)
    cov = jnp.mean(am * bm)
    sx = jnp.sqrt(jnp.mean(am * am))
    sy = jnp.sqrt(jnp.mean(bm * bm))
    corr = cov / (sx * sy)
    out_ref[...] = jnp.abs(jnp.clip(corr, -1.0, 1.0)).reshape(1, 1)


def kernel(y_true, y_pred):
    yt = jnp.transpose(y_true[..., 0], (0, 1, 3, 2))
    yp = jnp.transpose(y_pred[..., 0], (0, 1, 3, 2))
    acc1, acc2, cnt = pl.pallas_call(
        _sums_body,
        grid=(_NB,),
        in_specs=[
            pl.BlockSpec((1, 1, _D, _W), lambda b: (b, 0, 0, 0)),
            pl.BlockSpec((1, 1, _D, _W), lambda b: (b + _NB, 0, 0, 0)),
            pl.BlockSpec((1, _H, _D, _W), lambda b: (b, 0, 0, 0)),
            pl.BlockSpec((1, _H, _D, _W), lambda b: (b + _NB, 0, 0, 0)),
        ],
        out_specs=[
            pl.BlockSpec((_H, _W), lambda b: (0, 0)),
            pl.BlockSpec((_H, _W), lambda b: (0, 0)),
            pl.BlockSpec((1, 128), lambda b: (0, 0)),
        ],
        out_shape=[
            jax.ShapeDtypeStruct((_H, _W), jnp.float32),
            jax.ShapeDtypeStruct((_H, _W), jnp.float32),
            jax.ShapeDtypeStruct((1, 128), jnp.float32),
        ],
    )(yt, yt, yp, yp)
    out = pl.pallas_call(
        _corr_body,
        out_shape=jax.ShapeDtypeStruct((1, 1), jnp.float32),
    )(acc1, acc2, cnt)
    return out
